# baseline (speedup 1.0000x reference)
import functools

import jax
import jax.numpy as jnp
from jax.experimental import pallas as pl
from jax.experimental.pallas import tpu as pltpu

CTX = 256
N = 20000
BLOCK_N = 2000
NB = N // BLOCK_N


def _bbox_kernel(x_ref, s_ref, w_ref, bias_ref, out_ref, off_ref, idx_ref,
                 idx_smem):
    nb = pl.program_id(1)

    @pl.when(nb == 0)
    def _():
        s = s_ref[0]
        m = jnp.max(s)
        ii = jax.lax.broadcasted_iota(jnp.int32, s.shape, 1)
        idx = jnp.min(jnp.where(s == m, ii, N))
        idx_smem[0] = idx
        idx_ref[...] = jnp.full((1, 1, 1), idx, jnp.int32)

    x = x_ref[0].astype(jnp.bfloat16)
    y = jnp.dot(x, w_ref[...], preferred_element_type=jnp.float32)
    y = y + bias_ref[...]
    out_ref[0] = y

    local = idx_smem[0] - nb * BLOCK_N

    @pl.when((local >= 0) & (local < BLOCK_N))
    def _():
        rows = jax.lax.broadcasted_iota(jnp.int32, (BLOCK_N, 1), 0)
        off_ref[0] = jnp.sum(jnp.where(rows == local, y, 0.0), axis=0,
                             keepdims=True)


@functools.partial(jax.jit, static_argnames=())
def kernel(x_out, ref_scores, W, b):
    B = x_out.shape[0]
    bias = b.reshape(1, 4)
    out, off, idx = pl.pallas_call(
        _bbox_kernel,
        grid=(B, NB),
        in_specs=[
            pl.BlockSpec((1, BLOCK_N, CTX), lambda bi, nb: (bi, nb, 0)),
            pl.BlockSpec((1, 1, N), lambda bi, nb: (bi, 0, 0)),
            pl.BlockSpec((CTX, 4), lambda bi, nb: (0, 0)),
            pl.BlockSpec((1, 4), lambda bi, nb: (0, 0)),
        ],
        out_specs=[
            pl.BlockSpec((1, BLOCK_N, 4), lambda bi, nb: (bi, nb, 0)),
            pl.BlockSpec((1, 1, 4), lambda bi, nb: (bi, 0, 0)),
            pl.BlockSpec((1, 1, 1), lambda bi, nb: (bi, 0, 0)),
        ],
        out_shape=[
            jax.ShapeDtypeStruct((B, N, 4), jnp.float32),
            jax.ShapeDtypeStruct((B, 1, 4), jnp.float32),
            jax.ShapeDtypeStruct((B, 1, 1), jnp.int32),
        ],
        scratch_shapes=[pltpu.SMEM((1,), jnp.int32)],
    )(x_out, ref_scores.reshape(B, 1, N), W.astype(jnp.bfloat16), bias)
    rows = jnp.arange(B, dtype=jnp.int32)
    slice_inds = jnp.stack([rows, idx.reshape(B)], axis=1)
    return (off.reshape(B, 4), out, slice_inds)


# speedup vs baseline: 1.1597x; 1.1597x over previous
import functools

import jax
import jax.numpy as jnp
from jax.experimental import pallas as pl
from jax.experimental.pallas import tpu as pltpu

CTX = 256
N = 20000
BLOCK_N = 4000
NB = N // BLOCK_N


def _bbox_kernel(x_ref, s_ref, w_ref, bias_ref, out_ref, off_ref, idx_ref,
                 idx_smem):
    nb = pl.program_id(1)

    @pl.when(nb == 0)
    def _():
        s = s_ref[0]
        m = jnp.max(s)
        ii = jax.lax.broadcasted_iota(jnp.int32, s.shape, 1)
        idx = jnp.min(jnp.where(s == m, ii, N))
        idx_smem[0] = idx
        idx_ref[...] = jnp.full((1, 1, 1), idx, jnp.int32)

    x = x_ref[0].astype(jnp.bfloat16)
    y = jnp.dot(x, w_ref[...], preferred_element_type=jnp.float32)
    y = y + bias_ref[...]
    out_ref[0] = y

    local = idx_smem[0] - nb * BLOCK_N

    @pl.when((local >= 0) & (local < BLOCK_N))
    def _():
        rows = jax.lax.broadcasted_iota(jnp.int32, (BLOCK_N, 1), 0)
        off_ref[0] = jnp.sum(jnp.where(rows == local, y, 0.0), axis=0,
                             keepdims=True)


@functools.partial(jax.jit, static_argnames=())
def kernel(x_out, ref_scores, W, b):
    B = x_out.shape[0]
    bias = b.reshape(1, 4)
    out, off, idx = pl.pallas_call(
        _bbox_kernel,
        grid=(B, NB),
        in_specs=[
            pl.BlockSpec((1, BLOCK_N, CTX), lambda bi, nb: (bi, nb, 0)),
            pl.BlockSpec((1, 1, N), lambda bi, nb: (bi, 0, 0)),
            pl.BlockSpec((CTX, 4), lambda bi, nb: (0, 0)),
            pl.BlockSpec((1, 4), lambda bi, nb: (0, 0)),
        ],
        out_specs=[
            pl.BlockSpec((1, BLOCK_N, 4), lambda bi, nb: (bi, nb, 0)),
            pl.BlockSpec((1, 1, 4), lambda bi, nb: (bi, 0, 0)),
            pl.BlockSpec((1, 1, 1), lambda bi, nb: (bi, 0, 0)),
        ],
        out_shape=[
            jax.ShapeDtypeStruct((B, N, 4), jnp.float32),
            jax.ShapeDtypeStruct((B, 1, 4), jnp.float32),
            jax.ShapeDtypeStruct((B, 1, 1), jnp.int32),
        ],
        scratch_shapes=[pltpu.SMEM((1,), jnp.int32)],
    )(x_out, ref_scores.reshape(B, 1, N), W.astype(jnp.bfloat16), bias)
    rows = jnp.arange(B, dtype=jnp.int32)
    slice_inds = jnp.stack([rows, idx.reshape(B)], axis=1)
    return (off.reshape(B, 4), out, slice_inds)
